# baseline (speedup 1.0000x reference)
import jax
import jax.numpy as jnp
from jax.experimental import pallas as pl
from jax.experimental.pallas import tpu as pltpu

TOKENS = 32
VOCAB = 1000000
EPS = 1e-10
CHUNK = 65536
GRID = (VOCAB + CHUNK - 1) // CHUNK
SUB = 8192
NSUB = CHUNK // SUB


def _phase1(x_ref, e_ref, t_ref, c_ref, m_ref):
    i = pl.program_id(0)

    @pl.when(i == 0)
    def _init():
        m_ref[...] = jnp.full((TOKENS, 1), -jnp.inf, jnp.float32)
        c_ref[...] = jnp.zeros((TOKENS, 1), jnp.int32)

    x = x_ref[...]
    e = e_ref[...]
    t = t_ref[...]

    noise = jnp.log(e + EPS)
    key = x - t * noise

    idx = jax.lax.broadcasted_iota(jnp.int32, key.shape, 1)
    key = jnp.where(idx < VOCAB - i * CHUNK, key, -jnp.inf)

    sub_max = jnp.max(key.reshape(TOKENS, NSUB, SUB), axis=2)
    loc_max = jnp.max(sub_max, axis=1, keepdims=True)
    sidx = jax.lax.broadcasted_iota(jnp.int32, sub_max.shape, 1)
    loc_sub = jnp.min(
        jnp.where(sub_max == loc_max, sidx, NSUB), axis=1, keepdims=True
    ) + i * NSUB

    better = loc_max > m_ref[...]
    m_ref[...] = jnp.where(better, loc_max, m_ref[...])
    c_ref[...] = jnp.where(better, loc_sub, c_ref[...])


def _phase2(sid_ref, tsm_ref, x_ref, e_ref, o_ref):
    r = pl.program_id(0)
    x = x_ref[...]
    e = e_ref[...]
    t = tsm_ref[r]
    sid = sid_ref[r]

    noise = jnp.log(e + EPS)
    key = x.reshape(1, SUB) - t * noise

    idx = jax.lax.broadcasted_iota(jnp.int32, key.shape, 1)
    key = jnp.where(idx < VOCAB - sid * SUB, key, -jnp.inf)

    loc_max = jnp.max(key)
    arg = jnp.min(jnp.where(key == loc_max, idx, SUB)) + sid * SUB
    o_ref[...] = jnp.reshape(arg, (1, 1, 1))


@jax.jit
def kernel(logits, temperatures, exponential):
    t = temperatures[:, None].astype(jnp.float32)

    sub_ids = pl.pallas_call(
        _phase1,
        grid=(GRID,),
        in_specs=[
            pl.BlockSpec((TOKENS, CHUNK), lambda i: (0, i)),
            pl.BlockSpec((1, CHUNK), lambda i: (0, i)),
            pl.BlockSpec((TOKENS, 1), lambda i: (0, 0)),
        ],
        out_specs=pl.BlockSpec((TOKENS, 1), lambda i: (0, 0)),
        out_shape=jax.ShapeDtypeStruct((TOKENS, 1), jnp.int32),
        scratch_shapes=[pltpu.VMEM((TOKENS, 1), jnp.float32)],
    )(logits, exponential, t)

    sub_ids_flat = sub_ids[:, 0]

    grid_spec = pltpu.PrefetchScalarGridSpec(
        num_scalar_prefetch=2,
        grid=(TOKENS,),
        in_specs=[
            pl.BlockSpec((1, 1, SUB), lambda r, sid, tsm: (r, 0, sid[r])),
            pl.BlockSpec((1, SUB), lambda r, sid, tsm: (0, sid[r])),
        ],
        out_specs=pl.BlockSpec((1, 1, 1), lambda r, sid, tsm: (r, 0, 0)),
    )
    out = pl.pallas_call(
        _phase2,
        grid_spec=grid_spec,
        out_shape=jax.ShapeDtypeStruct((TOKENS, 1, 1), jnp.int32),
    )(sub_ids_flat, temperatures.astype(jnp.float32),
      logits.reshape(TOKENS, 1, VOCAB), exponential)
    return out[:, 0, 0]


# speedup vs baseline: 2.2404x; 2.2404x over previous
import jax
import jax.numpy as jnp
from jax.experimental import pallas as pl
from jax.experimental.pallas import tpu as pltpu

TOKENS = 32
VOCAB = 1000000
EPS = 1e-10
CHUNK = 65536
GRID = (VOCAB + CHUNK - 1) // CHUNK
SUB = 8192
NSUB = CHUNK // SUB


def _phase1(x_ref, e_ref, t_ref, c_ref, m_ref):
    i = pl.program_id(0)

    @pl.when(i == 0)
    def _init():
        m_ref[...] = jnp.full((TOKENS, 1), -jnp.inf, jnp.float32)
        c_ref[...] = jnp.zeros((TOKENS, 1), jnp.int32)

    x = x_ref[...]
    e = e_ref[...]
    t = t_ref[...]

    noise = jnp.log(e + EPS)
    key = x - t * noise

    idx = jax.lax.broadcasted_iota(jnp.int32, key.shape, 1)
    key = jnp.where(idx < VOCAB - i * CHUNK, key, -jnp.inf)

    sub_max = jnp.max(key.reshape(TOKENS, NSUB, SUB), axis=2)
    loc_max = jnp.max(sub_max, axis=1, keepdims=True)
    sidx = jax.lax.broadcasted_iota(jnp.int32, sub_max.shape, 1)
    loc_sub = jnp.min(
        jnp.where(sub_max == loc_max, sidx, NSUB), axis=1, keepdims=True
    ) + i * NSUB

    better = loc_max > m_ref[...]
    m_ref[...] = jnp.where(better, loc_max, m_ref[...])
    c_ref[...] = jnp.where(better, loc_sub, c_ref[...])


def _phase2(sid_ref, tsm_ref, x_ref, e_ref, o_ref):
    r = pl.program_id(0)
    x = x_ref[...]
    e = e_ref[...]
    t = tsm_ref[r]
    sid = sid_ref[r]

    noise = jnp.log(e + EPS)
    key = x - t * noise

    row = jax.lax.broadcasted_iota(jnp.int32, key.shape, 0)
    idx = jax.lax.broadcasted_iota(jnp.int32, key.shape, 1)
    key = jnp.where((row == r % 8) & (idx < VOCAB - sid * SUB), key, -jnp.inf)

    loc_max = jnp.max(key)
    arg = jnp.min(jnp.where(key == loc_max, idx, SUB)) + sid * SUB
    o_ref[r] = arg


@jax.jit
def kernel(logits, temperatures, exponential):
    t = temperatures[:, None].astype(jnp.float32)

    sub_ids = pl.pallas_call(
        _phase1,
        grid=(GRID,),
        in_specs=[
            pl.BlockSpec((TOKENS, CHUNK), lambda i: (0, i)),
            pl.BlockSpec((1, CHUNK), lambda i: (0, i)),
            pl.BlockSpec((TOKENS, 1), lambda i: (0, 0)),
        ],
        out_specs=pl.BlockSpec((TOKENS, 1), lambda i: (0, 0)),
        out_shape=jax.ShapeDtypeStruct((TOKENS, 1), jnp.int32),
        scratch_shapes=[pltpu.VMEM((TOKENS, 1), jnp.float32)],
    )(logits, exponential, t)

    sub_ids_flat = sub_ids[:, 0]

    grid_spec = pltpu.PrefetchScalarGridSpec(
        num_scalar_prefetch=2,
        grid=(TOKENS,),
        in_specs=[
            pl.BlockSpec((8, SUB), lambda r, sid, tsm: (r // 8, sid[r])),
            pl.BlockSpec((1, SUB), lambda r, sid, tsm: (0, sid[r])),
        ],
        out_specs=pl.BlockSpec(memory_space=pltpu.SMEM),
    )
    out = pl.pallas_call(
        _phase2,
        grid_spec=grid_spec,
        out_shape=jax.ShapeDtypeStruct((TOKENS,), jnp.int32),
    )(sub_ids_flat, temperatures.astype(jnp.float32), logits, exponential)
    return out
